# Initial kernel scaffold; baseline (speedup 1.0000x reference)
import jax
import jax.numpy as jnp
from jax.experimental import pallas as pl


def kernel(data, matrix, conv1_W, conv1_b, conv2_W, conv2_b, pool_W, pool_b, lin1_W, lin1_b, lin2_W, lin2_b):
    raise NotImplementedError("write your pallas kernel here")


# speedup vs baseline: 3.3060x; 3.3060x over previous
import functools

import jax
import jax.numpy as jnp
import numpy as np
from jax.experimental import pallas as pl

_N = 512
_D = 256
_F1 = 128
_H = 512
_O = 64
_R = 8

_DS = np.float32(2.0 ** -0.5)
_DS2 = _DS * _DS


def _prologue_kernel(data_ref, mf_ref, w1_ref, b1_ref, w2_ref,
                     u_ref, v_ref, hc2_ref, ide_ref, cc_ref, keepf_ref):
    data = data_ref[...]
    mf = mf_ref[...]
    h1 = jax.lax.dot_general(data, w1_ref[...], (((1,), (0,)), ((), ())),
                             preferred_element_type=jnp.float32)
    b1 = b1_ref[...]
    deg = 1.0 + jnp.sum(mf, axis=1, keepdims=True)
    dc = jax.lax.rsqrt(deg)
    ide = dc * dc
    cc = _DS * dc
    w_edge = mf * cc
    agg1 = jax.lax.dot_general(w_edge, h1, (((1,), (0,)), ((), ())),
                               preferred_element_type=jnp.float32)
    oc1 = h1 * ide + agg1 + b1
    xc1 = jnp.maximum(oc1, 0.0)
    hc2 = jax.lax.dot_general(xc1, w2_ref[...], (((1,), (0,)), ((), ())),
                              preferred_element_type=jnp.float32)
    u_ref[...] = _DS2 * h1 + b1
    v_ref[...] = cc * h1
    hc2_ref[...] = hc2
    ide_ref[...] = ide
    cc_ref[...] = cc
    keepf_ref[...] = jnp.floor((deg + 4.0) / 5.0)


def _main_kernel(u_ref, w2_ref, b2_ref, pw_ref, pb_ref,
                 l1wa_ref, l1wb_ref, l1b_ref, l2w_ref, l2b_ref,
                 v_ref, hc2_ref, mf_ref, ide_ref, cc_ref, keepf_ref,
                 out_ref):
    f32 = jnp.float32
    u = u_ref[...]
    v = v_ref[...]
    mf = mf_ref[...]
    ide = ide_ref[...]
    cc = cc_ref[...]
    hc2 = hc2_ref[...]
    b2 = b2_ref[...]
    pw = pw_ref[...]
    pb = pb_ref[0, 0]

    B = jnp.maximum(u[None, :, :] + v[:, None, :], 0.0)
    H2 = jax.lax.dot_general(
        B.reshape(_R * _N, _F1), w2_ref[...], (((1,), (0,)), ((), ())),
        preferred_element_type=f32).reshape(_R, _N, _H)

    w_edge = mf * cc
    aggB = jnp.sum(w_edge[:, :, None] * B, axis=1)
    agg2 = jax.lax.dot_general(aggB, w2_ref[...], (((1,), (0,)), ((), ())),
                               preferred_element_type=f32)
    xc2 = jnp.maximum(hc2 * ide + agg2 + b2, 0.0)

    xs2 = jnp.maximum(_DS2 * H2 + (cc * hc2)[:, None, :] + b2[None, :, :], 0.0)

    ps_in = jax.lax.dot_general(
        xs2.reshape(_R * _N, _H), pw, (((1,), (0,)), ((), ())),
        preferred_element_type=f32).reshape(_R, _N)
    pc_in = jax.lax.dot_general(xc2, pw, (((1,), (0,)), ((), ())),
                                preferred_element_type=f32)
    spin = jnp.sum(w_edge * ps_in, axis=1, keepdims=True)
    score_c = jnp.tanh(pc_in * ide + spin + pb)
    score_s = jnp.tanh(_DS2 * ps_in + cc * pc_in + pb)

    valid = mf > 0.0
    key_c = 1.0 - score_c
    key_s = jnp.where(valid, 1.0 - score_s, jnp.inf)
    ku_c = jax.lax.bitcast_convert_type(key_c, jnp.int32)
    ku_s = jax.lax.bitcast_convert_type(key_s, jnp.int32)
    k = keepf_ref[...].astype(jnp.int32)

    prefix = jnp.zeros((_R, 1), jnp.int32)
    for b in range(30, -1, -1):
        t = prefix | (1 << b)
        cnt = (jnp.sum((ku_s < t).astype(jnp.int32), axis=1, keepdims=True)
               + (ku_c < t).astype(jnp.int32))
        prefix = jnp.where(cnt < k, t, prefix)
    vk = prefix

    lt_c = ku_c < vk
    eq_c = ku_c == vk
    lt_s = ku_s < vk
    eq_s = ku_s == vk
    c_lt = (jnp.sum(lt_s.astype(jnp.int32), axis=1, keepdims=True)
            + lt_c.astype(jnp.int32))
    m = k - c_lt
    sel_c = lt_c | eq_c
    m_s = m - eq_c.astype(jnp.int32)
    rows = jax.lax.broadcasted_iota(jnp.int32, (_N, _N), 0)
    cols = jax.lax.broadcasted_iota(jnp.int32, (_N, _N), 1)
    tri = (rows <= cols).astype(f32)
    cum = jax.lax.dot_general(eq_s.astype(f32), tri, (((1,), (0,)), ((), ())),
                              preferred_element_type=f32)
    sel_s = lt_s | (eq_s & (cum <= m_s.astype(f32)))

    xp_c = xc2 * score_c
    xp_s = xs2 * score_s[:, :, None]
    neg = f32(-jnp.inf)
    sel3 = sel_s.astype(f32)[:, :, None] > 0.0
    gmp = jnp.max(jnp.where(sel3, xp_s, neg), axis=1)
    gmp = jnp.maximum(gmp, jnp.where(sel_c, xp_c, neg))
    gap = (jnp.sum(jnp.where(sel3, xp_s, 0.0), axis=1)
           + jnp.where(sel_c, xp_c, 0.0)) / keepf_ref[...]

    t1 = jnp.maximum(
        jax.lax.dot_general(gmp, l1wa_ref[...], (((1,), (0,)), ((), ())),
                            preferred_element_type=f32)
        + jax.lax.dot_general(gap, l1wb_ref[...], (((1,), (0,)), ((), ())),
                              preferred_element_type=f32)
        + l1b_ref[...], 0.0)
    out_ref[...] = (
        jax.lax.dot_general(t1, l2w_ref[...], (((1,), (0,)), ((), ())),
                            preferred_element_type=f32) + l2b_ref[...])


@jax.jit
def kernel(data, matrix, conv1_W, conv1_b, conv2_W, conv2_b, pool_W, pool_b,
           lin1_W, lin1_b, lin2_W, lin2_b):
    mf = (matrix != 0).astype(jnp.float32)

    u, v, hc2, ide, cc, keepf = pl.pallas_call(
        _prologue_kernel,
        out_shape=[
            jax.ShapeDtypeStruct((_N, _F1), jnp.float32),
            jax.ShapeDtypeStruct((_N, _F1), jnp.float32),
            jax.ShapeDtypeStruct((_N, _H), jnp.float32),
            jax.ShapeDtypeStruct((_N, 1), jnp.float32),
            jax.ShapeDtypeStruct((_N, 1), jnp.float32),
            jax.ShapeDtypeStruct((_N, 1), jnp.float32),
        ],
    )(data, mf, conv1_W, conv1_b.reshape(1, _F1), conv2_W)

    grid = (_N // _R,)
    full = lambda i: (0, 0)
    row = lambda i: (i, 0)
    out = pl.pallas_call(
        _main_kernel,
        grid=grid,
        in_specs=[
            pl.BlockSpec((_N, _F1), full),
            pl.BlockSpec((_F1, _H), full),
            pl.BlockSpec((1, _H), full),
            pl.BlockSpec((_H, 1), full),
            pl.BlockSpec((1, 1), full),
            pl.BlockSpec((_H, _H), full),
            pl.BlockSpec((_H, _H), full),
            pl.BlockSpec((1, _H), full),
            pl.BlockSpec((_H, _O), full),
            pl.BlockSpec((1, _O), full),
            pl.BlockSpec((_R, _F1), row),
            pl.BlockSpec((_R, _H), row),
            pl.BlockSpec((_R, _N), row),
            pl.BlockSpec((_R, 1), row),
            pl.BlockSpec((_R, 1), row),
            pl.BlockSpec((_R, 1), row),
        ],
        out_specs=pl.BlockSpec((_R, _O), row),
        out_shape=jax.ShapeDtypeStruct((_N, _O), jnp.float32),
    )(u, conv2_W, conv2_b.reshape(1, _H), pool_W, pool_b.reshape(1, 1),
      lin1_W[:_H], lin1_W[_H:], lin1_b.reshape(1, _H), lin2_W,
      lin2_b.reshape(1, _O), v, hc2, mf, ide, cc, keepf)
    return out


# speedup vs baseline: 3.5159x; 1.0635x over previous
import functools

import jax
import jax.numpy as jnp
import numpy as np
from jax.experimental import pallas as pl

_N = 512
_D = 256
_F1 = 128
_H = 512
_O = 64
_R = 8

_DS = np.float32(2.0 ** -0.5)
_DS2 = _DS * _DS


def _prologue_kernel(data_ref, mf_ref, w1_ref, b1_ref, w2_ref,
                     u_ref, v_ref, hc2_ref, ide_ref, cc_ref, keepf_ref):
    data = data_ref[...]
    mf = mf_ref[...]
    h1 = jax.lax.dot_general(data, w1_ref[...], (((1,), (0,)), ((), ())),
                             preferred_element_type=jnp.float32)
    b1 = b1_ref[...]
    deg = 1.0 + jnp.sum(mf, axis=1, keepdims=True)
    dc = jax.lax.rsqrt(deg)
    ide = dc * dc
    cc = _DS * dc
    w_edge = mf * cc
    agg1 = jax.lax.dot_general(w_edge, h1, (((1,), (0,)), ((), ())),
                               preferred_element_type=jnp.float32)
    oc1 = h1 * ide + agg1 + b1
    xc1 = jnp.maximum(oc1, 0.0)
    hc2 = jax.lax.dot_general(xc1, w2_ref[...], (((1,), (0,)), ((), ())),
                              preferred_element_type=jnp.float32)
    u_ref[...] = _DS2 * h1 + b1
    v_ref[...] = cc * h1
    hc2_ref[...] = hc2
    ide_ref[...] = ide
    cc_ref[...] = cc
    keepf_ref[...] = jnp.floor((deg + 4.0) / 5.0)


def _main_kernel(u_ref, w2_ref, b2_ref, pw_ref, pb_ref,
                 l1wa_ref, l1wb_ref, l1b_ref, l2w_ref, l2b_ref,
                 v_ref, hc2_ref, mf_ref, ide_ref, cc_ref, keepf_ref,
                 out_ref):
    f32 = jnp.float32
    u = u_ref[...]
    v = v_ref[...]
    mf = mf_ref[...]
    ide = ide_ref[...]
    cc = cc_ref[...]
    hc2 = hc2_ref[...]
    b2 = b2_ref[...]
    pw = pw_ref[...]
    pb = pb_ref[0, 0]

    B = jnp.maximum(u[None, :, :] + v[:, None, :], 0.0)
    H2 = jax.lax.dot_general(
        B.reshape(_R * _N, _F1), w2_ref[...], (((1,), (0,)), ((), ())),
        preferred_element_type=f32).reshape(_R, _N, _H)

    w_edge = mf * cc
    agg2 = jax.lax.dot_general(w_edge, H2, (((1,), (1,)), ((0,), (0,))),
                               preferred_element_type=f32)
    xc2 = jnp.maximum(hc2 * ide + agg2 + b2, 0.0)

    xs2 = jnp.maximum(_DS2 * H2 + (cc * hc2)[:, None, :] + b2[None, :, :], 0.0)

    ps_in = jax.lax.dot_general(
        xs2.reshape(_R * _N, _H), pw, (((1,), (0,)), ((), ())),
        preferred_element_type=f32).reshape(_R, _N)
    pc_in = jax.lax.dot_general(xc2, pw, (((1,), (0,)), ((), ())),
                                preferred_element_type=f32)
    spin = jax.lax.dot_general(
        w_edge, ps_in.reshape(_R, _N, 1), (((1,), (1,)), ((0,), (0,))),
        preferred_element_type=f32)
    score_c = jnp.tanh(pc_in * ide + spin + pb)
    score_s = jnp.tanh(_DS2 * ps_in + cc * pc_in + pb)

    valid = mf > 0.0
    key_c = 1.0 - score_c
    key_s = jnp.where(valid, 1.0 - score_s, jnp.inf)
    ku_c = jax.lax.bitcast_convert_type(key_c, jnp.int32)
    ku_s = jax.lax.bitcast_convert_type(key_s, jnp.int32)
    k = keepf_ref[...].astype(jnp.int32)

    prefix = jnp.zeros((_R, 1), jnp.int32)
    for b in range(30, -1, -1):
        t = prefix | (1 << b)
        cnt = (jnp.sum((ku_s < t).astype(jnp.int32), axis=1, keepdims=True)
               + (ku_c < t).astype(jnp.int32))
        prefix = jnp.where(cnt < k, t, prefix)
    vk = prefix

    lt_c = ku_c < vk
    eq_c = ku_c == vk
    lt_s = ku_s < vk
    eq_s = ku_s == vk
    c_lt = (jnp.sum(lt_s.astype(jnp.int32), axis=1, keepdims=True)
            + lt_c.astype(jnp.int32))
    m = k - c_lt
    sel_c = lt_c | eq_c
    m_s = m - eq_c.astype(jnp.int32)
    rows = jax.lax.broadcasted_iota(jnp.int32, (_N, _N), 0)
    cols = jax.lax.broadcasted_iota(jnp.int32, (_N, _N), 1)
    tri = (rows <= cols).astype(f32)
    cum = jax.lax.dot_general(eq_s.astype(f32), tri, (((1,), (0,)), ((), ())),
                              preferred_element_type=f32)
    sel_s = lt_s | (eq_s & (cum <= m_s.astype(f32)))

    xp_c = xc2 * score_c
    sel_f = sel_s.astype(f32)
    neg = f32(-jnp.inf)
    sel3 = sel_f[:, :, None] > 0.0
    gmp = jnp.max(jnp.where(sel3, xs2 * score_s[:, :, None], neg), axis=1)
    gmp = jnp.maximum(gmp, jnp.where(sel_c, xp_c, neg))
    gap = (jax.lax.dot_general(sel_f * score_s, xs2,
                               (((1,), (1,)), ((0,), (0,))),
                               preferred_element_type=f32)
           + jnp.where(sel_c, xp_c, 0.0)) / keepf_ref[...]

    t1 = jnp.maximum(
        jax.lax.dot_general(gmp, l1wa_ref[...], (((1,), (0,)), ((), ())),
                            preferred_element_type=f32)
        + jax.lax.dot_general(gap, l1wb_ref[...], (((1,), (0,)), ((), ())),
                              preferred_element_type=f32)
        + l1b_ref[...], 0.0)
    out_ref[...] = (
        jax.lax.dot_general(t1, l2w_ref[...], (((1,), (0,)), ((), ())),
                            preferred_element_type=f32) + l2b_ref[...])


@jax.jit
def kernel(data, matrix, conv1_W, conv1_b, conv2_W, conv2_b, pool_W, pool_b,
           lin1_W, lin1_b, lin2_W, lin2_b):
    mf = (matrix != 0).astype(jnp.float32)

    u, v, hc2, ide, cc, keepf = pl.pallas_call(
        _prologue_kernel,
        out_shape=[
            jax.ShapeDtypeStruct((_N, _F1), jnp.float32),
            jax.ShapeDtypeStruct((_N, _F1), jnp.float32),
            jax.ShapeDtypeStruct((_N, _H), jnp.float32),
            jax.ShapeDtypeStruct((_N, 1), jnp.float32),
            jax.ShapeDtypeStruct((_N, 1), jnp.float32),
            jax.ShapeDtypeStruct((_N, 1), jnp.float32),
        ],
    )(data, mf, conv1_W, conv1_b.reshape(1, _F1), conv2_W)

    grid = (_N // _R,)
    full = lambda i: (0, 0)
    row = lambda i: (i, 0)
    out = pl.pallas_call(
        _main_kernel,
        grid=grid,
        in_specs=[
            pl.BlockSpec((_N, _F1), full),
            pl.BlockSpec((_F1, _H), full),
            pl.BlockSpec((1, _H), full),
            pl.BlockSpec((_H, 1), full),
            pl.BlockSpec((1, 1), full),
            pl.BlockSpec((_H, _H), full),
            pl.BlockSpec((_H, _H), full),
            pl.BlockSpec((1, _H), full),
            pl.BlockSpec((_H, _O), full),
            pl.BlockSpec((1, _O), full),
            pl.BlockSpec((_R, _F1), row),
            pl.BlockSpec((_R, _H), row),
            pl.BlockSpec((_R, _N), row),
            pl.BlockSpec((_R, 1), row),
            pl.BlockSpec((_R, 1), row),
            pl.BlockSpec((_R, 1), row),
        ],
        out_specs=pl.BlockSpec((_R, _O), row),
        out_shape=jax.ShapeDtypeStruct((_N, _O), jnp.float32),
    )(u, conv2_W, conv2_b.reshape(1, _H), pool_W, pool_b.reshape(1, 1),
      lin1_W[:_H], lin1_W[_H:], lin1_b.reshape(1, _H), lin2_W,
      lin2_b.reshape(1, _O), v, hc2, mf, ide, cc, keepf)
    return out


# speedup vs baseline: 4.7617x; 1.3543x over previous
import functools

import jax
import jax.numpy as jnp
import numpy as np
from jax.experimental import pallas as pl

_N = 512
_D = 256
_F1 = 128
_H = 512
_O = 64
_R = 8

_DS = np.float32(2.0 ** -0.5)
_DS2 = _DS * _DS


def _prologue_kernel(data_ref, mf_ref, w1_ref, b1_ref, w2_ref,
                     u_ref, v_ref, hc2_ref, ide_ref, cc_ref, keepf_ref):
    data = data_ref[...]
    mf = mf_ref[...]
    h1 = jax.lax.dot_general(data, w1_ref[...], (((1,), (0,)), ((), ())),
                             preferred_element_type=jnp.float32)
    b1 = b1_ref[...]
    deg = 1.0 + jnp.sum(mf, axis=1, keepdims=True)
    dc = jax.lax.rsqrt(deg)
    ide = dc * dc
    cc = _DS * dc
    w_edge = mf * cc
    agg1 = jax.lax.dot_general(w_edge, h1, (((1,), (0,)), ((), ())),
                               preferred_element_type=jnp.float32)
    oc1 = h1 * ide + agg1 + b1
    xc1 = jnp.maximum(oc1, 0.0)
    hc2 = jax.lax.dot_general(xc1, w2_ref[...], (((1,), (0,)), ((), ())),
                              preferred_element_type=jnp.float32)
    u_ref[...] = _DS2 * h1 + b1
    v_ref[...] = cc * h1
    hc2_ref[...] = hc2
    ide_ref[...] = ide
    cc_ref[...] = cc
    keepf_ref[...] = jnp.floor((deg + 4.0) / 5.0)


def _main_kernel(u_ref, w2_ref, b2_ref, pw_ref, pb_ref,
                 l1wa_ref, l1wb_ref, l1b_ref, l2w_ref, l2b_ref,
                 v_ref, hc2_ref, mf_ref, ide_ref, cc_ref, keepf_ref,
                 out_ref):
    f32 = jnp.float32
    u = u_ref[...]
    v = v_ref[...]
    mf = mf_ref[...]
    ide = ide_ref[...]
    cc = cc_ref[...]
    hc2 = hc2_ref[...]
    b2 = b2_ref[...]
    pw = pw_ref[...]
    pb = pb_ref[0, 0]

    B = jnp.maximum(u[None, :, :] + v[:, None, :], 0.0)
    H2 = jax.lax.dot_general(
        B.reshape(_R * _N, _F1), w2_ref[...], (((1,), (0,)), ((), ())),
        preferred_element_type=f32).reshape(_R, _N, _H)

    w_edge = mf * cc
    agg2 = jax.lax.dot_general(w_edge, H2, (((1,), (1,)), ((0,), (0,))),
                               preferred_element_type=f32)
    xc2 = jnp.maximum(hc2 * ide + agg2 + b2, 0.0)

    xs2 = jnp.maximum(_DS2 * H2 + (cc * hc2)[:, None, :] + b2[None, :, :], 0.0)

    ps_in = jax.lax.dot_general(
        xs2.reshape(_R * _N, _H), pw, (((1,), (0,)), ((), ())),
        preferred_element_type=f32).reshape(_R, _N)
    pc_in = jax.lax.dot_general(xc2, pw, (((1,), (0,)), ((), ())),
                                preferred_element_type=f32)
    spin = jax.lax.dot_general(
        w_edge, ps_in.reshape(_R, _N, 1), (((1,), (1,)), ((0,), (0,))),
        preferred_element_type=f32)
    score_c = jnp.tanh(pc_in * ide + spin + pb)
    score_s = jnp.tanh(_DS2 * ps_in + cc * pc_in + pb)

    valid = mf > 0.0
    key_c = 1.0 - score_c
    key_s = jnp.where(valid, 1.0 - score_s, jnp.inf)
    ku_c = jax.lax.bitcast_convert_type(key_c, jnp.int32)
    ku_s = jax.lax.bitcast_convert_type(key_s, jnp.int32)
    k = keepf_ref[...].astype(jnp.int32)

    prefix = jnp.zeros((_R, 1), jnp.int32)
    for shift, mmax in ((28, 7), (24, 15), (20, 15), (16, 15),
                        (12, 15), (8, 15), (4, 15), (0, 15)):
        nibble = jnp.zeros((_R, 1), jnp.int32)
        for m in range(1, mmax + 1):
            t = prefix | (m << shift)
            cnt = (jnp.sum((ku_s < t).astype(jnp.int32), axis=1, keepdims=True)
                   + (ku_c < t).astype(jnp.int32))
            nibble += (cnt < k).astype(jnp.int32)
        prefix = prefix | (nibble << shift)
    vk = prefix

    lt_c = ku_c < vk
    eq_c = ku_c == vk
    lt_s = ku_s < vk
    eq_s = ku_s == vk
    c_lt = (jnp.sum(lt_s.astype(jnp.int32), axis=1, keepdims=True)
            + lt_c.astype(jnp.int32))
    m = k - c_lt
    sel_c = lt_c | eq_c
    m_s = m - eq_c.astype(jnp.int32)
    rows = jax.lax.broadcasted_iota(jnp.int32, (_N, _N), 0)
    cols = jax.lax.broadcasted_iota(jnp.int32, (_N, _N), 1)
    tri = (rows <= cols).astype(f32)
    cum = jax.lax.dot_general(eq_s.astype(f32), tri, (((1,), (0,)), ((), ())),
                              preferred_element_type=f32)
    sel_s = lt_s | (eq_s & (cum <= m_s.astype(f32)))

    xp_c = xc2 * score_c
    sel_f = sel_s.astype(f32)
    neg = f32(-jnp.inf)
    sel3 = sel_f[:, :, None] > 0.0
    gmp = jnp.max(jnp.where(sel3, xs2 * score_s[:, :, None], neg), axis=1)
    gmp = jnp.maximum(gmp, jnp.where(sel_c, xp_c, neg))
    gap = (jax.lax.dot_general(sel_f * score_s, xs2,
                               (((1,), (1,)), ((0,), (0,))),
                               preferred_element_type=f32)
           + jnp.where(sel_c, xp_c, 0.0)) / keepf_ref[...]

    t1 = jnp.maximum(
        jax.lax.dot_general(gmp, l1wa_ref[...], (((1,), (0,)), ((), ())),
                            preferred_element_type=f32)
        + jax.lax.dot_general(gap, l1wb_ref[...], (((1,), (0,)), ((), ())),
                              preferred_element_type=f32)
        + l1b_ref[...], 0.0)
    out_ref[...] = (
        jax.lax.dot_general(t1, l2w_ref[...], (((1,), (0,)), ((), ())),
                            preferred_element_type=f32) + l2b_ref[...])


@jax.jit
def kernel(data, matrix, conv1_W, conv1_b, conv2_W, conv2_b, pool_W, pool_b,
           lin1_W, lin1_b, lin2_W, lin2_b):
    mf = (matrix != 0).astype(jnp.float32)

    u, v, hc2, ide, cc, keepf = pl.pallas_call(
        _prologue_kernel,
        out_shape=[
            jax.ShapeDtypeStruct((_N, _F1), jnp.float32),
            jax.ShapeDtypeStruct((_N, _F1), jnp.float32),
            jax.ShapeDtypeStruct((_N, _H), jnp.float32),
            jax.ShapeDtypeStruct((_N, 1), jnp.float32),
            jax.ShapeDtypeStruct((_N, 1), jnp.float32),
            jax.ShapeDtypeStruct((_N, 1), jnp.float32),
        ],
    )(data, mf, conv1_W, conv1_b.reshape(1, _F1), conv2_W)

    grid = (_N // _R,)
    full = lambda i: (0, 0)
    row = lambda i: (i, 0)
    out = pl.pallas_call(
        _main_kernel,
        grid=grid,
        in_specs=[
            pl.BlockSpec((_N, _F1), full),
            pl.BlockSpec((_F1, _H), full),
            pl.BlockSpec((1, _H), full),
            pl.BlockSpec((_H, 1), full),
            pl.BlockSpec((1, 1), full),
            pl.BlockSpec((_H, _H), full),
            pl.BlockSpec((_H, _H), full),
            pl.BlockSpec((1, _H), full),
            pl.BlockSpec((_H, _O), full),
            pl.BlockSpec((1, _O), full),
            pl.BlockSpec((_R, _F1), row),
            pl.BlockSpec((_R, _H), row),
            pl.BlockSpec((_R, _N), row),
            pl.BlockSpec((_R, 1), row),
            pl.BlockSpec((_R, 1), row),
            pl.BlockSpec((_R, 1), row),
        ],
        out_specs=pl.BlockSpec((_R, _O), row),
        out_shape=jax.ShapeDtypeStruct((_N, _O), jnp.float32),
    )(u, conv2_W, conv2_b.reshape(1, _H), pool_W, pool_b.reshape(1, 1),
      lin1_W[:_H], lin1_W[_H:], lin1_b.reshape(1, _H), lin2_W,
      lin2_b.reshape(1, _O), v, hc2, mf, ide, cc, keepf)
    return out


# speedup vs baseline: 4.8334x; 1.0151x over previous
import functools

import jax
import jax.numpy as jnp
import numpy as np
from jax.experimental import pallas as pl

_N = 512
_D = 256
_F1 = 128
_H = 512
_O = 64
_R = 8

_DS = np.float32(2.0 ** -0.5)
_DS2 = _DS * _DS


def _prologue_kernel(data_ref, mf_ref, w1_ref, b1_ref, w2_ref,
                     u_ref, v_ref, hc2_ref, ide_ref, cc_ref, keepf_ref):
    data = data_ref[...]
    mf = mf_ref[...]
    h1 = jax.lax.dot_general(data, w1_ref[...], (((1,), (0,)), ((), ())),
                             preferred_element_type=jnp.float32)
    b1 = b1_ref[...]
    deg = 1.0 + jnp.sum(mf, axis=1, keepdims=True)
    dc = jax.lax.rsqrt(deg)
    ide = dc * dc
    cc = _DS * dc
    w_edge = mf * cc
    agg1 = jax.lax.dot_general(w_edge, h1, (((1,), (0,)), ((), ())),
                               preferred_element_type=jnp.float32)
    oc1 = h1 * ide + agg1 + b1
    xc1 = jnp.maximum(oc1, 0.0)
    hc2 = jax.lax.dot_general(xc1, w2_ref[...], (((1,), (0,)), ((), ())),
                              preferred_element_type=jnp.float32)
    u_ref[...] = _DS2 * h1 + b1
    v_ref[...] = cc * h1
    hc2_ref[...] = hc2
    ide_ref[...] = ide
    cc_ref[...] = cc
    keepf_ref[...] = jnp.floor((deg + 4.0) / 5.0)


def _main_kernel(u_ref, w2_ref, b2_ref, pw_ref, pb_ref,
                 l1wa_ref, l1wb_ref, l1b_ref, l2w_ref, l2b_ref,
                 v_ref, hc2_ref, mf_ref, ide_ref, cc_ref, keepf_ref,
                 out_ref):
    f32 = jnp.float32
    u = u_ref[...]
    v = v_ref[...]
    mf = mf_ref[...]
    ide = ide_ref[...]
    cc = cc_ref[...]
    hc2 = hc2_ref[...]
    b2 = b2_ref[...]
    pw = pw_ref[...]
    pb = pb_ref[0, 0]

    B = jnp.maximum(u[None, :, :] + v[:, None, :], 0.0)
    H2 = jax.lax.dot_general(
        B.reshape(_R * _N, _F1), w2_ref[...], (((1,), (0,)), ((), ())),
        preferred_element_type=f32).reshape(_R, _N, _H)

    w_edge = mf * cc
    agg2 = jax.lax.dot_general(w_edge, H2, (((1,), (1,)), ((0,), (0,))),
                               preferred_element_type=f32)
    xc2 = jnp.maximum(hc2 * ide + agg2 + b2, 0.0)

    xs2 = jnp.maximum(_DS2 * H2 + (cc * hc2)[:, None, :] + b2[None, :, :], 0.0)

    ps_in = jax.lax.dot_general(
        xs2.reshape(_R * _N, _H), pw, (((1,), (0,)), ((), ())),
        preferred_element_type=f32).reshape(_R, _N)
    pc_in = jax.lax.dot_general(xc2, pw, (((1,), (0,)), ((), ())),
                                preferred_element_type=f32)
    spin = jax.lax.dot_general(
        w_edge, ps_in.reshape(_R, _N, 1), (((1,), (1,)), ((0,), (0,))),
        preferred_element_type=f32)
    score_c = jnp.tanh(pc_in * ide + spin + pb)
    score_s = jnp.tanh(_DS2 * ps_in + cc * pc_in + pb)

    valid = mf > 0.0
    key_c = 1.0 - score_c
    key_s = jnp.where(valid, 1.0 - score_s, jnp.inf)
    ku_c = jax.lax.bitcast_convert_type(key_c, jnp.int32)
    ku_s = jax.lax.bitcast_convert_type(key_s, jnp.int32)
    k = keepf_ref[...].astype(jnp.int32)

    prefix = jnp.zeros((_R, 1), jnp.int32)
    for shift, mmax in ((28, 7), (24, 15), (20, 15), (16, 15),
                        (12, 15), (8, 15), (4, 15), (0, 15)):
        nibble = jnp.zeros((_R, 1), jnp.int32)
        for m in range(1, mmax + 1):
            t = prefix | (m << shift)
            cnt = (jnp.sum((ku_s < t).astype(jnp.int32), axis=1, keepdims=True)
                   + (ku_c < t).astype(jnp.int32))
            nibble += (cnt < k).astype(jnp.int32)
        prefix = prefix | (nibble << shift)
    vk = prefix

    lt_c = ku_c < vk
    eq_c = ku_c == vk
    lt_s = ku_s < vk
    eq_s = ku_s == vk
    c_lt = (jnp.sum(lt_s.astype(jnp.int32), axis=1, keepdims=True)
            + lt_c.astype(jnp.int32))
    m = k - c_lt
    sel_c = lt_c | eq_c
    m_s = m - eq_c.astype(jnp.int32)
    rows = jax.lax.broadcasted_iota(jnp.int32, (_N, _N), 0)
    cols = jax.lax.broadcasted_iota(jnp.int32, (_N, _N), 1)
    tri = (rows <= cols).astype(f32)
    cum = jax.lax.dot_general(eq_s.astype(f32), tri, (((1,), (0,)), ((), ())),
                              preferred_element_type=f32)
    sel_s = lt_s | (eq_s & (cum <= m_s.astype(f32)))

    xp_c = xc2 * score_c
    sel_f = sel_s.astype(f32)
    neg = f32(-jnp.inf)
    wsel = sel_f * score_s
    offs = (sel_f - 1.0) * f32(3e38)
    gmp = jnp.max(wsel[:, :, None] * xs2 + offs[:, :, None], axis=1)
    gmp = jnp.maximum(gmp, jnp.where(sel_c, xp_c, neg))
    gap = (jax.lax.dot_general(wsel, xs2,
                               (((1,), (1,)), ((0,), (0,))),
                               preferred_element_type=f32)
           + jnp.where(sel_c, xp_c, 0.0)) / keepf_ref[...]

    t1 = jnp.maximum(
        jax.lax.dot_general(gmp, l1wa_ref[...], (((1,), (0,)), ((), ())),
                            preferred_element_type=f32)
        + jax.lax.dot_general(gap, l1wb_ref[...], (((1,), (0,)), ((), ())),
                              preferred_element_type=f32)
        + l1b_ref[...], 0.0)
    out_ref[...] = (
        jax.lax.dot_general(t1, l2w_ref[...], (((1,), (0,)), ((), ())),
                            preferred_element_type=f32) + l2b_ref[...])


@jax.jit
def kernel(data, matrix, conv1_W, conv1_b, conv2_W, conv2_b, pool_W, pool_b,
           lin1_W, lin1_b, lin2_W, lin2_b):
    mf = (matrix != 0).astype(jnp.float32)

    u, v, hc2, ide, cc, keepf = pl.pallas_call(
        _prologue_kernel,
        out_shape=[
            jax.ShapeDtypeStruct((_N, _F1), jnp.float32),
            jax.ShapeDtypeStruct((_N, _F1), jnp.float32),
            jax.ShapeDtypeStruct((_N, _H), jnp.float32),
            jax.ShapeDtypeStruct((_N, 1), jnp.float32),
            jax.ShapeDtypeStruct((_N, 1), jnp.float32),
            jax.ShapeDtypeStruct((_N, 1), jnp.float32),
        ],
    )(data, mf, conv1_W, conv1_b.reshape(1, _F1), conv2_W)

    grid = (_N // _R,)
    full = lambda i: (0, 0)
    row = lambda i: (i, 0)
    out = pl.pallas_call(
        _main_kernel,
        grid=grid,
        in_specs=[
            pl.BlockSpec((_N, _F1), full),
            pl.BlockSpec((_F1, _H), full),
            pl.BlockSpec((1, _H), full),
            pl.BlockSpec((_H, 1), full),
            pl.BlockSpec((1, 1), full),
            pl.BlockSpec((_H, _H), full),
            pl.BlockSpec((_H, _H), full),
            pl.BlockSpec((1, _H), full),
            pl.BlockSpec((_H, _O), full),
            pl.BlockSpec((1, _O), full),
            pl.BlockSpec((_R, _F1), row),
            pl.BlockSpec((_R, _H), row),
            pl.BlockSpec((_R, _N), row),
            pl.BlockSpec((_R, 1), row),
            pl.BlockSpec((_R, 1), row),
            pl.BlockSpec((_R, 1), row),
        ],
        out_specs=pl.BlockSpec((_R, _O), row),
        out_shape=jax.ShapeDtypeStruct((_N, _O), jnp.float32),
    )(u, conv2_W, conv2_b.reshape(1, _H), pool_W, pool_b.reshape(1, 1),
      lin1_W[:_H], lin1_W[_H:], lin1_b.reshape(1, _H), lin2_W,
      lin2_b.reshape(1, _O), v, hc2, mf, ide, cc, keepf)
    return out


# speedup vs baseline: 4.8592x; 1.0053x over previous
import functools

import jax
import jax.numpy as jnp
import numpy as np
from jax.experimental import pallas as pl
from jax.experimental.pallas import tpu as pltpu

_N = 512
_D = 256
_F1 = 128
_H = 512
_O = 64
_R = 8

_DS = np.float32(2.0 ** -0.5)
_DS2 = _DS * _DS


def _prologue_kernel(data_ref, mf_ref, w1_ref, b1_ref, w2_ref,
                     u_ref, v_ref, hc2_ref, ide_ref, cc_ref, keepf_ref):
    data = data_ref[...]
    mf = mf_ref[...]
    h1 = jax.lax.dot_general(data, w1_ref[...], (((1,), (0,)), ((), ())),
                             preferred_element_type=jnp.float32)
    b1 = b1_ref[...]
    deg = 1.0 + jnp.sum(mf, axis=1, keepdims=True)
    dc = jax.lax.rsqrt(deg)
    ide = dc * dc
    cc = _DS * dc
    w_edge = mf * cc
    agg1 = jax.lax.dot_general(w_edge, h1, (((1,), (0,)), ((), ())),
                               preferred_element_type=jnp.float32)
    oc1 = h1 * ide + agg1 + b1
    xc1 = jnp.maximum(oc1, 0.0)
    hc2 = jax.lax.dot_general(xc1, w2_ref[...], (((1,), (0,)), ((), ())),
                              preferred_element_type=jnp.float32)
    u_ref[...] = _DS2 * h1 + b1
    v_ref[...] = cc * h1
    hc2_ref[...] = hc2
    ide_ref[...] = ide
    cc_ref[...] = cc
    keepf_ref[...] = jnp.floor((deg + 4.0) / 5.0)


def _main_kernel(u_ref, w2_ref, b2_ref, pw_ref, pb_ref,
                 l1wa_ref, l1wb_ref, l1b_ref, l2w_ref, l2b_ref,
                 v_ref, hc2_ref, mf_ref, ide_ref, cc_ref,
                 mfp_ref, keepfp_ref,
                 out_ref,
                 xs2_scr, scs_scr, kus_scr, kuc_scr, xpc_scr):
    f32 = jnp.float32
    t = pl.program_id(0)
    p = jax.lax.rem(t, 2)
    q = 1 - p

    ku_s = kus_scr[q]
    ku_c = kuc_scr[q]
    score_s = scs_scr[q]
    xp_c = xpc_scr[q]
    keepf = keepfp_ref[...]
    k = keepf.astype(jnp.int32)

    prefix = jnp.zeros((_R, 1), jnp.int32)
    for shift, mmax in ((28, 7), (24, 15), (20, 15), (16, 15),
                        (12, 15), (8, 15), (4, 15), (0, 15)):
        nibble = jnp.zeros((_R, 1), jnp.int32)
        for m in range(1, mmax + 1):
            tt = prefix | (m << shift)
            cnt = (jnp.sum((ku_s < tt).astype(jnp.int32), axis=1, keepdims=True)
                   + (ku_c < tt).astype(jnp.int32))
            nibble += (cnt < k).astype(jnp.int32)
        prefix = prefix | (nibble << shift)
    vk = prefix

    lt_c = ku_c < vk
    eq_c = ku_c == vk
    lt_s = ku_s < vk
    eq_s = ku_s == vk
    c_lt = (jnp.sum(lt_s.astype(jnp.int32), axis=1, keepdims=True)
            + lt_c.astype(jnp.int32))
    m_sel = k - c_lt
    sel_c = lt_c | eq_c
    m_s = m_sel - eq_c.astype(jnp.int32)
    rows = jax.lax.broadcasted_iota(jnp.int32, (_N, _N), 0)
    cols = jax.lax.broadcasted_iota(jnp.int32, (_N, _N), 1)
    tri = (rows <= cols).astype(f32)
    cum = jax.lax.dot_general(eq_s.astype(f32), tri, (((1,), (0,)), ((), ())),
                              preferred_element_type=f32)
    sel_s = lt_s | (eq_s & (cum <= m_s.astype(f32)))

    xs2p = xs2_scr[q]
    sel_f = sel_s.astype(f32)
    neg = f32(-jnp.inf)
    wsel = sel_f * score_s
    offs = (sel_f - 1.0) * f32(3e38)
    gmp = jnp.max(wsel[:, :, None] * xs2p + offs[:, :, None], axis=1)
    gmp = jnp.maximum(gmp, jnp.where(sel_c, xp_c, neg))
    gap = (jax.lax.dot_general(wsel, xs2p, (((1,), (1,)), ((0,), (0,))),
                               preferred_element_type=f32)
           + jnp.where(sel_c, xp_c, 0.0)) / keepf

    t1 = jnp.maximum(
        jax.lax.dot_general(gmp, l1wa_ref[...], (((1,), (0,)), ((), ())),
                            preferred_element_type=f32)
        + jax.lax.dot_general(gap, l1wb_ref[...], (((1,), (0,)), ((), ())),
                              preferred_element_type=f32)
        + l1b_ref[...], 0.0)
    out_ref[...] = (
        jax.lax.dot_general(t1, l2w_ref[...], (((1,), (0,)), ((), ())),
                            preferred_element_type=f32) + l2b_ref[...])

    u = u_ref[...]
    v = v_ref[...]
    mf = mf_ref[...]
    ide = ide_ref[...]
    cc = cc_ref[...]
    hc2 = hc2_ref[...]
    b2 = b2_ref[...]
    pw = pw_ref[...]
    pb = pb_ref[0, 0]

    B = jnp.maximum(u[None, :, :] + v[:, None, :], 0.0)
    H2 = jax.lax.dot_general(
        B.reshape(_R * _N, _F1), w2_ref[...], (((1,), (0,)), ((), ())),
        preferred_element_type=f32).reshape(_R, _N, _H)

    w_edge = mf * cc
    agg2 = jax.lax.dot_general(w_edge, H2, (((1,), (1,)), ((0,), (0,))),
                               preferred_element_type=f32)
    xc2 = jnp.maximum(hc2 * ide + agg2 + b2, 0.0)

    xs2 = jnp.maximum(_DS2 * H2 + (cc * hc2)[:, None, :] + b2[None, :, :], 0.0)

    ps_in = jax.lax.dot_general(
        xs2.reshape(_R * _N, _H), pw, (((1,), (0,)), ((), ())),
        preferred_element_type=f32).reshape(_R, _N)
    pc_in = jax.lax.dot_general(xc2, pw, (((1,), (0,)), ((), ())),
                                preferred_element_type=f32)
    spin = jax.lax.dot_general(
        w_edge, ps_in.reshape(_R, _N, 1), (((1,), (1,)), ((0,), (0,))),
        preferred_element_type=f32)
    score_c = jnp.tanh(pc_in * ide + spin + pb)
    score_s_new = jnp.tanh(_DS2 * ps_in + cc * pc_in + pb)

    valid = mf > 0.0
    key_c = 1.0 - score_c
    key_s = jnp.where(valid, 1.0 - score_s_new, jnp.inf)

    xs2_scr[p] = xs2
    scs_scr[p] = score_s_new
    kus_scr[p] = jax.lax.bitcast_convert_type(key_s, jnp.int32)
    kuc_scr[p] = jax.lax.bitcast_convert_type(key_c, jnp.int32)
    xpc_scr[p] = xc2 * score_c


@jax.jit
def kernel(data, matrix, conv1_W, conv1_b, conv2_W, conv2_b, pool_W, pool_b,
           lin1_W, lin1_b, lin2_W, lin2_b):
    mf = (matrix != 0).astype(jnp.float32)

    u, v, hc2, ide, cc, keepf = pl.pallas_call(
        _prologue_kernel,
        out_shape=[
            jax.ShapeDtypeStruct((_N, _F1), jnp.float32),
            jax.ShapeDtypeStruct((_N, _F1), jnp.float32),
            jax.ShapeDtypeStruct((_N, _H), jnp.float32),
            jax.ShapeDtypeStruct((_N, 1), jnp.float32),
            jax.ShapeDtypeStruct((_N, 1), jnp.float32),
            jax.ShapeDtypeStruct((_N, 1), jnp.float32),
        ],
    )(data, mf, conv1_W, conv1_b.reshape(1, _F1), conv2_W)

    nblk = _N // _R
    grid = (nblk + 1,)
    full = lambda t: (0, 0)
    cur = lambda t: (jnp.minimum(t, nblk - 1), 0)
    prev = lambda t: (jnp.maximum(t - 1, 0), 0)
    out = pl.pallas_call(
        _main_kernel,
        grid=grid,
        in_specs=[
            pl.BlockSpec((_N, _F1), full),
            pl.BlockSpec((_F1, _H), full),
            pl.BlockSpec((1, _H), full),
            pl.BlockSpec((_H, 1), full),
            pl.BlockSpec((1, 1), full),
            pl.BlockSpec((_H, _H), full),
            pl.BlockSpec((_H, _H), full),
            pl.BlockSpec((1, _H), full),
            pl.BlockSpec((_H, _O), full),
            pl.BlockSpec((1, _O), full),
            pl.BlockSpec((_R, _F1), cur),
            pl.BlockSpec((_R, _H), cur),
            pl.BlockSpec((_R, _N), cur),
            pl.BlockSpec((_R, 1), cur),
            pl.BlockSpec((_R, 1), cur),
            pl.BlockSpec((_R, _N), prev),
            pl.BlockSpec((_R, 1), prev),
        ],
        out_specs=pl.BlockSpec((_R, _O), prev),
        out_shape=jax.ShapeDtypeStruct((_N, _O), jnp.float32),
        scratch_shapes=[
            pltpu.VMEM((2, _R, _N, _H), jnp.float32),
            pltpu.VMEM((2, _R, _N), jnp.float32),
            pltpu.VMEM((2, _R, _N), jnp.int32),
            pltpu.VMEM((2, _R, 1), jnp.int32),
            pltpu.VMEM((2, _R, _H), jnp.float32),
        ],
    )(u, conv2_W, conv2_b.reshape(1, _H), pool_W, pool_b.reshape(1, 1),
      lin1_W[:_H], lin1_W[_H:], lin1_b.reshape(1, _H), lin2_W,
      lin2_b.reshape(1, _O), v, hc2, mf, ide, cc, mf, keepf)
    return out


# speedup vs baseline: 6.4813x; 1.3338x over previous
import functools

import jax
import jax.numpy as jnp
import numpy as np
from jax.experimental import pallas as pl
from jax.experimental.pallas import tpu as pltpu

_N = 512
_D = 256
_F1 = 128
_H = 512
_O = 64
_R = 16

_DS = np.float32(2.0 ** -0.5)
_DS2 = _DS * _DS


def _prologue_kernel(data_ref, mf_ref, w1_ref, b1_ref, w2_ref,
                     u_ref, v_ref, hc2_ref, ide_ref, cc_ref, keepf_ref):
    data = data_ref[...]
    mf = mf_ref[...]
    h1 = jax.lax.dot_general(data, w1_ref[...], (((1,), (0,)), ((), ())),
                             preferred_element_type=jnp.float32)
    b1 = b1_ref[...]
    deg = 1.0 + jnp.sum(mf, axis=1, keepdims=True)
    dc = jax.lax.rsqrt(deg)
    ide = dc * dc
    cc = _DS * dc
    w_edge = mf * cc
    agg1 = jax.lax.dot_general(w_edge, h1, (((1,), (0,)), ((), ())),
                               preferred_element_type=jnp.float32)
    oc1 = h1 * ide + agg1 + b1
    xc1 = jnp.maximum(oc1, 0.0)
    hc2 = jax.lax.dot_general(xc1, w2_ref[...], (((1,), (0,)), ((), ())),
                              preferred_element_type=jnp.float32)
    u_ref[...] = _DS2 * h1 + b1
    v_ref[...] = cc * h1
    hc2_ref[...] = hc2
    ide_ref[...] = ide
    cc_ref[...] = cc
    keepf_ref[...] = jnp.floor((deg + 4.0) / 5.0)


def _main_kernel(u_ref, w2_ref, b2_ref, pw_ref, pb_ref,
                 l1wa_ref, l1wb_ref, l1b_ref, l2w_ref, l2b_ref,
                 v_ref, hc2_ref, mf_ref, ide_ref, cc_ref,
                 mfp_ref, keepfp_ref,
                 out_ref,
                 xs2_scr, scs_scr, kus_scr, kuc_scr, xpc_scr):
    f32 = jnp.float32
    t = pl.program_id(0)
    p = jax.lax.rem(t, 2)
    q = 1 - p

    ku_s = kus_scr[q]
    ku_c = kuc_scr[q]
    score_s = scs_scr[q]
    xp_c = xpc_scr[q]
    keepf = keepfp_ref[...]
    k = keepf.astype(jnp.int32)

    prefix = jnp.zeros((_R, 1), jnp.int32)
    for shift, mmax in ((28, 7), (24, 15), (20, 15), (16, 15),
                        (12, 15), (8, 15), (4, 15), (0, 15)):
        nibble = jnp.zeros((_R, 1), jnp.int32)
        for m in range(1, mmax + 1):
            tt = prefix | (m << shift)
            cnt = (jnp.sum((ku_s < tt).astype(jnp.int32), axis=1, keepdims=True)
                   + (ku_c < tt).astype(jnp.int32))
            nibble += (cnt < k).astype(jnp.int32)
        prefix = prefix | (nibble << shift)
    vk = prefix

    lt_c = ku_c < vk
    eq_c = ku_c == vk
    lt_s = ku_s < vk
    eq_s = ku_s == vk
    c_lt = (jnp.sum(lt_s.astype(jnp.int32), axis=1, keepdims=True)
            + lt_c.astype(jnp.int32))
    m_sel = k - c_lt
    sel_c = lt_c | eq_c
    m_s = m_sel - eq_c.astype(jnp.int32)
    rows = jax.lax.broadcasted_iota(jnp.int32, (_N, _N), 0)
    cols = jax.lax.broadcasted_iota(jnp.int32, (_N, _N), 1)
    tri = (rows <= cols).astype(f32)
    cum = jax.lax.dot_general(eq_s.astype(f32), tri, (((1,), (0,)), ((), ())),
                              preferred_element_type=f32)
    sel_s = lt_s | (eq_s & (cum <= m_s.astype(f32)))

    xs2p = xs2_scr[q]
    sel_f = sel_s.astype(f32)
    neg = f32(-jnp.inf)
    wsel = sel_f * score_s
    bf16 = jnp.bfloat16
    wsel_b = wsel.astype(bf16)
    offs_b = ((sel_f - 1.0) * f32(3e38)).astype(bf16)
    gmp = jnp.max(wsel_b[:, :, None] * xs2p + offs_b[:, :, None],
                  axis=1).astype(f32)
    gmp = jnp.maximum(gmp, jnp.where(sel_c, xp_c, neg))
    gap = (jax.lax.dot_general(wsel_b, xs2p, (((1,), (1,)), ((0,), (0,))),
                               preferred_element_type=f32)
           + jnp.where(sel_c, xp_c, 0.0)) / keepf

    t1 = jnp.maximum(
        jax.lax.dot_general(gmp, l1wa_ref[...], (((1,), (0,)), ((), ())),
                            preferred_element_type=f32)
        + jax.lax.dot_general(gap, l1wb_ref[...], (((1,), (0,)), ((), ())),
                              preferred_element_type=f32)
        + l1b_ref[...], 0.0)
    out_ref[...] = (
        jax.lax.dot_general(t1, l2w_ref[...], (((1,), (0,)), ((), ())),
                            preferred_element_type=f32) + l2b_ref[...])

    u = u_ref[...]
    v = v_ref[...]
    mf = mf_ref[...]
    ide = ide_ref[...]
    cc = cc_ref[...]
    hc2 = hc2_ref[...]
    b2 = b2_ref[...]
    pw = pw_ref[...]
    pb = pb_ref[0, 0]

    B = jnp.maximum(u[None, :, :] + v[:, None, :], 0.0)
    H2s = jax.lax.dot_general(
        B.reshape(_R * _N, _F1), w2_ref[...], (((1,), (0,)), ((), ())),
        preferred_element_type=f32).reshape(_R, _N, _H)

    w_edge2 = mf * (cc * (1.0 / _DS2))
    agg2 = jax.lax.dot_general(w_edge2, H2s, (((1,), (1,)), ((0,), (0,))),
                               preferred_element_type=f32)
    xc2 = jnp.maximum(hc2 * ide + agg2 + b2, 0.0)

    d_add = cc * hc2 + b2
    xs2 = jnp.maximum(H2s + d_add[:, None, :], 0.0)

    ps_in = jax.lax.dot_general(
        pw.reshape(1, _H), xs2, (((1,), (2,)), ((), ())),
        preferred_element_type=f32)[0]
    pc_in = jax.lax.dot_general(xc2, pw, (((1,), (0,)), ((), ())),
                                preferred_element_type=f32)
    spin = jnp.sum(mf * ps_in, axis=1, keepdims=True) * cc
    score_c = jnp.tanh(pc_in * ide + spin + pb)
    score_s_new = jnp.tanh(_DS2 * ps_in + cc * pc_in + pb)

    valid = mf > 0.0
    key_c = 1.0 - score_c
    key_s = jnp.where(valid, 1.0 - score_s_new, jnp.inf)

    xs2_scr[p] = xs2.astype(jnp.bfloat16)
    scs_scr[p] = score_s_new
    kus_scr[p] = jax.lax.bitcast_convert_type(key_s, jnp.int32)
    kuc_scr[p] = jax.lax.bitcast_convert_type(key_c, jnp.int32)
    xpc_scr[p] = xc2 * score_c


@jax.jit
def kernel(data, matrix, conv1_W, conv1_b, conv2_W, conv2_b, pool_W, pool_b,
           lin1_W, lin1_b, lin2_W, lin2_b):
    mf = (matrix != 0).astype(jnp.float32)

    u, v, hc2, ide, cc, keepf = pl.pallas_call(
        _prologue_kernel,
        out_shape=[
            jax.ShapeDtypeStruct((_N, _F1), jnp.float32),
            jax.ShapeDtypeStruct((_N, _F1), jnp.float32),
            jax.ShapeDtypeStruct((_N, _H), jnp.float32),
            jax.ShapeDtypeStruct((_N, 1), jnp.float32),
            jax.ShapeDtypeStruct((_N, 1), jnp.float32),
            jax.ShapeDtypeStruct((_N, 1), jnp.float32),
        ],
    )(data, mf, conv1_W, conv1_b.reshape(1, _F1), conv2_W)

    nblk = _N // _R
    grid = (nblk + 1,)
    full = lambda t: (0, 0)
    cur = lambda t: (jnp.minimum(t, nblk - 1), 0)
    prev = lambda t: (jnp.maximum(t - 1, 0), 0)
    out = pl.pallas_call(
        _main_kernel,
        grid=grid,
        in_specs=[
            pl.BlockSpec((_N, _F1), full),
            pl.BlockSpec((_F1, _H), full),
            pl.BlockSpec((1, _H), full),
            pl.BlockSpec((_H, 1), full),
            pl.BlockSpec((1, 1), full),
            pl.BlockSpec((_H, _H), full),
            pl.BlockSpec((_H, _H), full),
            pl.BlockSpec((1, _H), full),
            pl.BlockSpec((_H, _O), full),
            pl.BlockSpec((1, _O), full),
            pl.BlockSpec((_R, _F1), cur),
            pl.BlockSpec((_R, _H), cur),
            pl.BlockSpec((_R, _N), cur),
            pl.BlockSpec((_R, 1), cur),
            pl.BlockSpec((_R, 1), cur),
            pl.BlockSpec((_R, _N), prev),
            pl.BlockSpec((_R, 1), prev),
        ],
        out_specs=pl.BlockSpec((_R, _O), prev),
        out_shape=jax.ShapeDtypeStruct((_N, _O), jnp.float32),
        scratch_shapes=[
            pltpu.VMEM((2, _R, _N, _H), jnp.bfloat16),
            pltpu.VMEM((2, _R, _N), jnp.float32),
            pltpu.VMEM((2, _R, _N), jnp.int32),
            pltpu.VMEM((2, _R, 1), jnp.int32),
            pltpu.VMEM((2, _R, _H), jnp.float32),
        ],
    )(u, conv2_W * _DS2, conv2_b.reshape(1, _H), pool_W, pool_b.reshape(1, 1),
      lin1_W[:_H], lin1_W[_H:], lin1_b.reshape(1, _H), lin2_W,
      lin2_b.reshape(1, _O), v, hc2, mf, ide, cc, mf, keepf)
    return out
